# baseline (speedup 1.0000x reference)
import jax
import jax.numpy as jnp
from jax import lax
from jax.experimental import pallas as pl
from jax.experimental.pallas import tpu as pltpu
from jax.experimental.pallas import tpu_sc as plsc

D_MODEL = 64
SCALE = 8.0
NC, NS = 2, 16
NW = NC * NS
SUB = 128
NSUB = 4
CHUNK = SUB * NSUB


def _encoder_body(src_hbm, table_hbm, out_hbm, idx_v, rows_v, sem):
    wid = lax.axis_index("s") * NC + lax.axis_index("c")
    n_chunks = (src_hbm.shape[0] * SUB) // (NW * CHUNK)
    base_chunk = wid * n_chunks

    @pl.loop(0, n_chunks)
    def _chunk(i):
        c = base_chunk + i
        pltpu.sync_copy(src_hbm.at[pl.ds(c * NSUB, NSUB)], idx_v)
        copies = [
            pltpu.async_copy(
                table_hbm.at[idx_v.at[s]], rows_v.at[pl.ds(s * SUB, SUB)], sem
            )
            for s in range(NSUB)
        ]
        for cp in copies:
            cp.wait()

        @pl.loop(0, CHUNK, unroll=4)
        def _scale(r):
            for j in range(D_MODEL // 16):
                sl = pl.ds(j * 16, 16)
                rows_v[r, sl] = rows_v[r, sl] * SCALE

        pltpu.sync_copy(rows_v, out_hbm.at[pl.ds(c * CHUNK, CHUNK)])


def kernel(src, table):
    b, h = src.shape
    n = b * h
    src2 = src.astype(jnp.int32).reshape(n // SUB, SUB)
    mesh = plsc.VectorSubcoreMesh(
        core_axis_name="c", subcore_axis_name="s", num_cores=NC, num_subcores=NS
    )
    out = pl.kernel(
        _encoder_body,
        out_type=jax.ShapeDtypeStruct((n, D_MODEL), jnp.float32),
        mesh=mesh,
        scratch_types=[
            pltpu.VMEM((NSUB, SUB), jnp.int32),
            pltpu.VMEM((CHUNK, D_MODEL), jnp.float32),
            pltpu.SemaphoreType.DMA,
        ],
        compiler_params=pltpu.CompilerParams(use_tc_tiling_on_sc=False),
    )(src2, table)
    return out.reshape(b, h, D_MODEL)


# speedup vs baseline: 1.0880x; 1.0880x over previous
import jax
import jax.numpy as jnp
from jax import lax
from jax.experimental import pallas as pl
from jax.experimental.pallas import tpu as pltpu
from jax.experimental.pallas import tpu_sc as plsc

D_MODEL = 64
SCALE = 8.0
NC, NS = 2, 16
NW = NC * NS
SUB = 128
NSUB = 4
CHUNK = SUB * NSUB


def _encoder_body(src_hbm, table_hbm, out_hbm, idx_all, rows2, sg0, sg1, ss0, ss1):
    wid = lax.axis_index("s") * NC + lax.axis_index("c")
    n_chunks = (src_hbm.shape[0] * SUB) // (NW * CHUNK)
    idx_rows = n_chunks * NSUB
    base = wid * n_chunks
    sg = (sg0, sg1)
    ss = (ss0, ss1)

    pltpu.sync_copy(src_hbm.at[pl.ds(wid * idx_rows, idx_rows)], idx_all)

    def fire_gather(i, b):
        for s in range(NSUB):
            pltpu.async_copy(
                table_hbm.at[idx_all.at[i * NSUB + s]],
                rows2.at[b, pl.ds(s * SUB, SUB)],
                sg[b],
            )

    def wait_gather(b):
        pltpu.make_async_copy(
            table_hbm.at[pl.ds(0, CHUNK)], rows2.at[b], sg[b]
        ).wait()

    def scale(b):
        @pl.loop(0, CHUNK, unroll=4)
        def _(r):
            for j in range(D_MODEL // 16):
                sl = pl.ds(j * 16, 16)
                rows2[b, r, sl] = rows2[b, r, sl] * SCALE

    def fire_store(i, b):
        pltpu.async_copy(
            rows2.at[b], out_hbm.at[pl.ds((base + i) * CHUNK, CHUNK)], ss[b]
        )

    def wait_store(b):
        pltpu.make_async_copy(
            rows2.at[b], out_hbm.at[pl.ds(0, CHUNK)], ss[b]
        ).wait()

    fire_gather(0, 0)
    fire_gather(1, 1)
    wait_gather(0)
    scale(0)
    fire_store(0, 0)

    @pl.loop(0, (n_chunks - 2) // 2)
    def _pair(k):
        i = 1 + 2 * k
        wait_store(0)
        fire_gather(i + 1, 0)
        wait_gather(1)
        scale(1)
        fire_store(i, 1)

        wait_store(1)
        fire_gather(i + 2, 1)
        wait_gather(0)
        scale(0)
        fire_store(i + 1, 0)

    wait_gather(1)
    scale(1)
    fire_store(n_chunks - 1, 1)
    wait_store(0)
    wait_store(1)


def kernel(src, table):
    b, h = src.shape
    n = b * h
    src2 = src.astype(jnp.int32).reshape(n // SUB, SUB)
    n_chunks_w = n // (NW * CHUNK)
    mesh = plsc.VectorSubcoreMesh(
        core_axis_name="c", subcore_axis_name="s", num_cores=NC, num_subcores=NS
    )
    out = pl.kernel(
        _encoder_body,
        out_type=jax.ShapeDtypeStruct((n, D_MODEL), jnp.float32),
        mesh=mesh,
        scratch_types=[
            pltpu.VMEM((n_chunks_w * NSUB, SUB), jnp.int32),
            pltpu.VMEM((2, CHUNK, D_MODEL), jnp.float32),
            pltpu.SemaphoreType.DMA,
            pltpu.SemaphoreType.DMA,
            pltpu.SemaphoreType.DMA,
            pltpu.SemaphoreType.DMA,
        ],
        compiler_params=pltpu.CompilerParams(use_tc_tiling_on_sc=False),
    )(src2, table)
    return out.reshape(b, h, D_MODEL)
